# Initial kernel scaffold; baseline (speedup 1.0000x reference)
import jax
import jax.numpy as jnp
from jax.experimental import pallas as pl


def kernel(deltas, class_logits, proposals):
    raise NotImplementedError("write your pallas kernel here")


# speedup vs baseline: 128.1593x; 128.1593x over previous
import functools
import jax
import jax.numpy as jnp
from jax import lax
from jax.experimental import pallas as pl
from jax.experimental.pallas import tpu as pltpu
from jax.experimental.pallas import tpu_sc as plsc

SCORE_THR = 0.05
IOU_THR = 0.3
MAX_OUT = 100
N = 5000
C = 21
B = 2
NC = 2
NS = 16
L = 16
NPAD = 5120
PT = NPAD // NS
G = PT // L
OUTP = 112
ZROW = B * NPAD


def _sc_body(del_h, prop_h, log_h, ltab_h, orow_h, gath_h,
             dl, pr, lg, by1, bx1, by2, bx2, bar, bcl, bsc,
             orow, lrow, idxb, tvec, rbuf, shared, sem):
    c = lax.axis_index("c")
    s = lax.axis_index("s")
    base = s * PT
    c_off = c * NPAD

    iota = lax.iota(jnp.int32, L)
    iotaf = iota.astype(jnp.float32)
    zv = jnp.zeros((L,), jnp.float32)
    zi = jnp.zeros((L,), jnp.int32)

    for k in range(4):
        pltpu.sync_copy(del_h.at[c * 4 + k, pl.ds(base, PT)], dl.at[k])
        pltpu.sync_copy(prop_h.at[c * 4 + k, pl.ds(base, PT)], pr.at[k])
    for k in range(C):
        pltpu.sync_copy(log_h.at[c * C + k, pl.ds(base, PT)], lg.at[k])

    def init(j, _):
        orow[pl.ds(j * L, L)] = zv
        return 0
    lax.fori_loop(0, (OUTP * L) // L, init, 0)

    def init2(j, _):
        idxb[pl.ds(j * L, L)] = zi + ZROW
        return 0
    lax.fori_loop(0, OUTP // L, init2, 0)

    def prep(g, _):
        sl = pl.ds(g * L, L)
        d0 = dl[0, sl]; d1 = dl[1, sl]; d2 = dl[2, sl]; d3 = dl[3, sl]
        p0 = pr[0, sl]; p1 = pr[1, sl]; p2 = pr[2, sl]; p3 = pr[3, sl]
        h = p2 - p0
        w = p3 - p1
        cy = p0 + 0.5 * h + d0 * h
        cx = p1 + 0.5 * w + d1 * w
        h2 = h * jnp.exp(d2)
        w2 = w * jnp.exp(d3)
        y1 = cy - 0.5 * h2
        x1 = cx - 0.5 * w2
        y2 = cy + 0.5 * h2
        x2 = cx + 0.5 * w2
        m1 = lg[1, sl]
        cls = zv + 1.0
        for cc in range(2, C):
            v = lg[cc, sl]
            upd = v > m1
            m1 = jnp.where(upd, v, m1)
            cls = jnp.where(upd, float(cc), cls)
        m_all = jnp.maximum(lg[0, sl], m1)
        z = zv
        for cc in range(C):
            z = z + jnp.exp(lg[cc, sl] - m_all)
        sc = jnp.exp(m1 - m_all) / z
        gi = base + g * L + iota
        ok = (sc > SCORE_THR) & (gi < N)
        by1[sl] = y1
        bx1[sl] = x1
        by2[sl] = y2
        bx2[sl] = x2
        bar[sl] = (y2 - y1) * (x2 - x1)
        bcl[sl] = cls
        bsc[sl] = jnp.where(ok, sc, -1.0)
        return 0

    lax.fori_loop(0, G, prep, 0)

    def nms_it(it, _):
        def amax(g, carry):
            bv, bi = carry
            v = bsc[pl.ds(g * L, L)]
            li = g * L + iota
            upd = v > bv
            return jnp.where(upd, v, bv), jnp.where(upd, li, bi)
        bv, bi = lax.fori_loop(0, G, amax,
                               (zv - 2.0, zi))
        m = jnp.max(bv)
        li = jnp.min(jnp.where(bv == m, bi, NPAD))
        liv = zi + li
        gvec = plsc.bitcast(liv + base, jnp.float32)
        slot = jnp.where(iota == 0, plsc.load_gather(bsc, [liv]),
               jnp.where(iota == 1, gvec,
               jnp.where(iota == 2, plsc.load_gather(by1, [liv]),
               jnp.where(iota == 3, plsc.load_gather(bx1, [liv]),
               jnp.where(iota == 4, plsc.load_gather(by2, [liv]),
               jnp.where(iota == 5, plsc.load_gather(bx2, [liv]),
                         plsc.load_gather(bcl, [liv])))))))
        tvec[...] = slot
        par = it & 1
        pltpu.sync_copy(tvec, shared.at[par, pl.ds(s * L, L)])
        plsc.subcore_barrier()
        pltpu.sync_copy(shared.at[par], rbuf)

        def fld(f):
            return plsc.load_gather(rbuf, [iota * L + f])
        vals = fld(0)
        gidxs = plsc.bitcast(fld(1), jnp.int32)
        gm = jnp.max(vals)
        cand = vals == gm
        widx = jnp.min(jnp.where(cand, gidxs, NPAD * NC))
        wsel = gidxs == widx
        wy1 = zv + jnp.sum(jnp.where(wsel, fld(2), 0.0))
        wx1 = zv + jnp.sum(jnp.where(wsel, fld(3), 0.0))
        wy2 = zv + jnp.sum(jnp.where(wsel, fld(4), 0.0))
        wx2 = zv + jnp.sum(jnp.where(wsel, fld(5), 0.0))
        wcl = zv + jnp.sum(jnp.where(wsel, fld(6), 0.0))
        gmv = zv + gm
        validv = gmv > 0.0
        warea = (wy2 - wy1) * (wx2 - wx1)
        widxv = zi + widx

        def supp(g, _):
            sl = pl.ds(g * L, L)
            y1 = by1[sl]; x1 = bx1[sl]; y2 = by2[sl]; x2 = bx2[sl]
            yy1 = jnp.maximum(wy1, y1)
            xx1 = jnp.maximum(wx1, x1)
            yy2 = jnp.minimum(wy2, y2)
            xx2 = jnp.minimum(wx2, x2)
            inter = jnp.maximum(yy2 - yy1, 0.0) * jnp.maximum(xx2 - xx1, 0.0)
            iou = inter / (warea + bar[sl] - inter + 1e-8)
            gidx = base + g * L + iota
            kill = ((bcl[sl] == wcl) & (iou > IOU_THR)) | (gidx == widxv)
            bsc[sl] = jnp.where(kill, -1.0, bsc[sl])
            return 0
        lax.fori_loop(0, G, supp, 0)

        tagv = jnp.where(validv, zv + 1.0, zv)
        gidxf = jnp.where(validv, widxv + c_off, zi + ZROW).astype(jnp.float32)
        row = jnp.where(iota == 0, jnp.where(validv, wy1, zv),
              jnp.where(iota == 1, jnp.where(validv, wx1, zv),
              jnp.where(iota == 2, jnp.where(validv, wy2, zv),
              jnp.where(iota == 3, jnp.where(validv, wx2, zv),
              jnp.where(iota == 4, tagv,
              jnp.where(iota == 5, jnp.where(validv, gmv, zv),
              jnp.where(iota == 6, tagv,
              jnp.where(iota == 7, jnp.where(validv, wcl, zv),
              jnp.where(iota == 8, tagv,
              jnp.where(iota == 9, gidxf, zv))))))))))
        orow[pl.ds(it * L, L)] = row
        return 0

    lax.fori_loop(0, MAX_OUT, nms_it, 0)

    @pl.when(s == 0)
    def _():
        for j in range(OUTP // L):
            gidx = plsc.load_gather(orow, [(j * L + iota) * L + 9])
            idxb[pl.ds(j * L, L)] = gidx.astype(jnp.int32)
        pltpu.async_copy(ltab_h.at[idxb], lrow, sem).wait()
        pltpu.sync_copy(orow, orow_h.at[c])
        pltpu.sync_copy(lrow, gath_h.at[c])


@functools.cache
def _build_sc_nms():
  mesh = plsc.VectorSubcoreMesh(core_axis_name="c", subcore_axis_name="s",
                                num_cores=NC, num_subcores=NS)
  return functools.partial(
    pl.kernel,
    out_type=(jax.ShapeDtypeStruct((B, OUTP * L), jnp.float32),
              jax.ShapeDtypeStruct((B, OUTP, 32), jnp.float32)),
    mesh=mesh,
    compiler_params=pltpu.CompilerParams(use_tc_tiling_on_sc=False,
                                         needs_layout_passes=False),
    scratch_types=[
        pltpu.VMEM((4, PT), jnp.float32),
        pltpu.VMEM((4, PT), jnp.float32),
        pltpu.VMEM((C, PT), jnp.float32),
        pltpu.VMEM((PT,), jnp.float32),
        pltpu.VMEM((PT,), jnp.float32),
        pltpu.VMEM((PT,), jnp.float32),
        pltpu.VMEM((PT,), jnp.float32),
        pltpu.VMEM((PT,), jnp.float32),
        pltpu.VMEM((PT,), jnp.float32),
        pltpu.VMEM((PT,), jnp.float32),
        pltpu.VMEM((OUTP * L,), jnp.float32),
        pltpu.VMEM((OUTP, 32), jnp.float32),
        pltpu.VMEM((OUTP,), jnp.int32),
        pltpu.VMEM((L,), jnp.float32),
        pltpu.VMEM((NS * L,), jnp.float32),
        pltpu.VMEM_SHARED((2, NS * L), jnp.float32),
        pltpu.SemaphoreType.DMA,
    ],
  )(_sc_body)


@jax.jit
def kernel(deltas, class_logits, proposals):
    pad_n = ((0, 0), (0, 0), (0, NPAD - N))
    d_t = jnp.pad(jnp.transpose(deltas, (0, 2, 1)), pad_n).reshape(B * 4, NPAD)
    p_t = jnp.pad(jnp.transpose(proposals[..., :4], (0, 2, 1)), pad_n).reshape(B * 4, NPAD)
    l_t = jnp.pad(jnp.transpose(class_logits, (0, 2, 1)), pad_n).reshape(B * C, NPAD)
    ltab = jnp.zeros((B * NPAD + 8, 32), jnp.float32)
    ltab = ltab.at[:B * NPAD, :C].set(
        jnp.pad(class_logits, ((0, 0), (0, NPAD - N), (0, 0))).reshape(B * NPAD, C))

    orow_o, gath_o = _build_sc_nms()(d_t, p_t, l_t, ltab)
    orow = orow_o.reshape(B, OUTP, L)
    boxes_out = orow[:, :MAX_OUT, 0:5]
    scores_out = orow[:, :MAX_OUT, 5:7]
    ids_out = orow[:, :MAX_OUT, 7:9].astype(jnp.int32)
    logits_out = jnp.concatenate(
        [gath_o[:, :MAX_OUT, :C], orow[:, :MAX_OUT, 4:5]], axis=-1)
    return boxes_out, scores_out, ids_out, logits_out


# speedup vs baseline: 141.3495x; 1.1029x over previous
import functools
import jax
import jax.numpy as jnp
from jax import lax
from jax.experimental import pallas as pl
from jax.experimental.pallas import tpu as pltpu
from jax.experimental.pallas import tpu_sc as plsc

SCORE_THR = 0.05
IOU_THR = 0.3
MAX_OUT = 100
N = 5000
C = 21
B = 2
NC = 2
NS = 16
L = 16
NPAD = 5120
PT = NPAD // NS
G = PT // L
OUTP = 112
ZROW = B * NPAD


def _sc_body(del_h, prop_h, log_h, ltab_h, orow_h, gath_h,
             dl, pr, lg, by1, bx1, by2, bx2, bar, bcl, bsc,
             orow, lrow, idxb, tvec, rbuf, shared, sem):
    c = lax.axis_index("c")
    s = lax.axis_index("s")
    base = s * PT
    c_off = c * NPAD

    iota = lax.iota(jnp.int32, L)
    iotaf = iota.astype(jnp.float32)
    zv = jnp.zeros((L,), jnp.float32)
    zi = jnp.zeros((L,), jnp.int32)

    for k in range(4):
        pltpu.sync_copy(del_h.at[c * 4 + k, pl.ds(base, PT)], dl.at[k])
        pltpu.sync_copy(prop_h.at[c * 4 + k, pl.ds(base, PT)], pr.at[k])
    for k in range(C):
        pltpu.sync_copy(log_h.at[c * C + k, pl.ds(base, PT)], lg.at[k])

    def init(j, _):
        orow[pl.ds(j * L, L)] = zv
        return 0
    lax.fori_loop(0, (OUTP * L) // L, init, 0)

    def init2(j, _):
        idxb[pl.ds(j * L, L)] = zi + ZROW
        return 0
    lax.fori_loop(0, OUTP // L, init2, 0)

    def prep(g, _):
        sl = pl.ds(g * L, L)
        d0 = dl[0, sl]; d1 = dl[1, sl]; d2 = dl[2, sl]; d3 = dl[3, sl]
        p0 = pr[0, sl]; p1 = pr[1, sl]; p2 = pr[2, sl]; p3 = pr[3, sl]
        h = p2 - p0
        w = p3 - p1
        cy = p0 + 0.5 * h + d0 * h
        cx = p1 + 0.5 * w + d1 * w
        h2 = h * jnp.exp(d2)
        w2 = w * jnp.exp(d3)
        y1 = cy - 0.5 * h2
        x1 = cx - 0.5 * w2
        y2 = cy + 0.5 * h2
        x2 = cx + 0.5 * w2
        m1 = lg[1, sl]
        cls = zv + 1.0
        for cc in range(2, C):
            v = lg[cc, sl]
            upd = v > m1
            m1 = jnp.where(upd, v, m1)
            cls = jnp.where(upd, float(cc), cls)
        m_all = jnp.maximum(lg[0, sl], m1)
        z = zv
        for cc in range(C):
            z = z + jnp.exp(lg[cc, sl] - m_all)
        sc = jnp.exp(m1 - m_all) / z
        gi = base + g * L + iota
        ok = (sc > SCORE_THR) & (gi < N)
        by1[sl] = y1
        bx1[sl] = x1
        by2[sl] = y2
        bx2[sl] = x2
        bar[sl] = (y2 - y1) * (x2 - x1)
        bcl[sl] = cls
        bsc[sl] = jnp.where(ok, sc, -1.0)
        return 0

    lax.fori_loop(0, G, prep, 0)

    def nms_it(it, carry):
        wy1, wx1, wy2, wx2, wcl, warea, widxv = carry

        def pass1(g, acc):
            bv, bi = acc
            sl = pl.ds(g * L, L)
            y1 = by1[sl]; x1 = bx1[sl]; y2 = by2[sl]; x2 = bx2[sl]
            yy1 = jnp.maximum(wy1, y1)
            xx1 = jnp.maximum(wx1, x1)
            yy2 = jnp.minimum(wy2, y2)
            xx2 = jnp.minimum(wx2, x2)
            inter = jnp.maximum(yy2 - yy1, 0.0) * jnp.maximum(xx2 - xx1, 0.0)
            iou = inter / (warea + bar[sl] - inter + 1e-8)
            lidx = g * L + iota
            kill = ((bcl[sl] == wcl) & (iou > IOU_THR)) | (lidx + base == widxv)
            sc2 = jnp.where(kill, -1.0, bsc[sl])
            bsc[sl] = sc2
            upd = sc2 > bv
            return jnp.where(upd, sc2, bv), jnp.where(upd, lidx, bi)

        bv, bi = lax.fori_loop(0, G, pass1, (zv - 2.0, zi), unroll=4)
        m = jnp.max(bv)
        liv = zi + jnp.min(jnp.where(bv == m, bi, NPAD))
        gvec = plsc.bitcast(liv + base, jnp.float32)
        slot = jnp.where(iota == 0, zv + m,
               jnp.where(iota == 1, gvec,
               jnp.where(iota == 2, plsc.load_gather(by1, [liv]),
               jnp.where(iota == 3, plsc.load_gather(bx1, [liv]),
               jnp.where(iota == 4, plsc.load_gather(by2, [liv]),
               jnp.where(iota == 5, plsc.load_gather(bx2, [liv]),
                         plsc.load_gather(bcl, [liv])))))))
        tvec[...] = slot
        par = it & 1
        pltpu.sync_copy(tvec, shared.at[par, pl.ds(s * L, L)])
        plsc.subcore_barrier()
        pltpu.sync_copy(shared.at[par], rbuf)

        vals = plsc.load_gather(rbuf, [iota * L])
        gm = jnp.max(vals)
        wlane = zi + plsc.all_reduce_ffs(vals == gm)

        def fw(f):
            return plsc.load_gather(rbuf, [wlane * L + f])

        nwidx = plsc.bitcast(fw(1), jnp.int32)
        ny1 = fw(2); nx1 = fw(3); ny2 = fw(4); nx2 = fw(5); ncl = fw(6)
        gmv = zv + gm
        validv = gmv > 0.0
        narea = (ny2 - ny1) * (nx2 - nx1)

        tagv = jnp.where(validv, zv + 1.0, zv)
        gidxf = jnp.where(validv, nwidx + c_off, zi + ZROW).astype(jnp.float32)
        row = jnp.where(iota == 0, jnp.where(validv, ny1, zv),
              jnp.where(iota == 1, jnp.where(validv, nx1, zv),
              jnp.where(iota == 2, jnp.where(validv, ny2, zv),
              jnp.where(iota == 3, jnp.where(validv, nx2, zv),
              jnp.where(iota == 4, tagv,
              jnp.where(iota == 5, jnp.where(validv, gmv, zv),
              jnp.where(iota == 6, tagv,
              jnp.where(iota == 7, jnp.where(validv, ncl, zv),
              jnp.where(iota == 8, tagv,
              jnp.where(iota == 9, gidxf, zv))))))))))
        orow[pl.ds(it * L, L)] = row
        return ny1, nx1, ny2, nx2, ncl, narea, nwidx

    lax.fori_loop(0, MAX_OUT, nms_it,
                  (zv, zv, zv, zv, zv - 1.0, zv, zi - 1))

    @pl.when(s == 0)
    def _():
        for j in range(OUTP // L):
            gidx = plsc.load_gather(orow, [(j * L + iota) * L + 9])
            idxb[pl.ds(j * L, L)] = gidx.astype(jnp.int32)
        pltpu.async_copy(ltab_h.at[idxb], lrow, sem).wait()
        pltpu.sync_copy(orow, orow_h.at[c])
        pltpu.sync_copy(lrow, gath_h.at[c])


@functools.cache
def _build_sc_nms():
  mesh = plsc.VectorSubcoreMesh(core_axis_name="c", subcore_axis_name="s",
                                num_cores=NC, num_subcores=NS)
  return functools.partial(
    pl.kernel,
    out_type=(jax.ShapeDtypeStruct((B, OUTP * L), jnp.float32),
              jax.ShapeDtypeStruct((B, OUTP, 32), jnp.float32)),
    mesh=mesh,
    compiler_params=pltpu.CompilerParams(use_tc_tiling_on_sc=False,
                                         needs_layout_passes=False),
    scratch_types=[
        pltpu.VMEM((4, PT), jnp.float32),
        pltpu.VMEM((4, PT), jnp.float32),
        pltpu.VMEM((C, PT), jnp.float32),
        pltpu.VMEM((PT,), jnp.float32),
        pltpu.VMEM((PT,), jnp.float32),
        pltpu.VMEM((PT,), jnp.float32),
        pltpu.VMEM((PT,), jnp.float32),
        pltpu.VMEM((PT,), jnp.float32),
        pltpu.VMEM((PT,), jnp.float32),
        pltpu.VMEM((PT,), jnp.float32),
        pltpu.VMEM((OUTP * L,), jnp.float32),
        pltpu.VMEM((OUTP, 32), jnp.float32),
        pltpu.VMEM((OUTP,), jnp.int32),
        pltpu.VMEM((L,), jnp.float32),
        pltpu.VMEM((NS * L,), jnp.float32),
        pltpu.VMEM_SHARED((2, NS * L), jnp.float32),
        pltpu.SemaphoreType.DMA,
    ],
  )(_sc_body)


@jax.jit
def kernel(deltas, class_logits, proposals):
    pad_n = ((0, 0), (0, 0), (0, NPAD - N))
    d_t = jnp.pad(jnp.transpose(deltas, (0, 2, 1)), pad_n).reshape(B * 4, NPAD)
    p_t = jnp.pad(jnp.transpose(proposals[..., :4], (0, 2, 1)), pad_n).reshape(B * 4, NPAD)
    l_t = jnp.pad(jnp.transpose(class_logits, (0, 2, 1)), pad_n).reshape(B * C, NPAD)
    ltab = jnp.zeros((B * NPAD + 8, 32), jnp.float32)
    ltab = ltab.at[:B * NPAD, :C].set(
        jnp.pad(class_logits, ((0, 0), (0, NPAD - N), (0, 0))).reshape(B * NPAD, C))

    orow_o, gath_o = _build_sc_nms()(d_t, p_t, l_t, ltab)
    orow = orow_o.reshape(B, OUTP, L)
    boxes_out = orow[:, :MAX_OUT, 0:5]
    scores_out = orow[:, :MAX_OUT, 5:7]
    ids_out = orow[:, :MAX_OUT, 7:9].astype(jnp.int32)
    logits_out = jnp.concatenate(
        [gath_o[:, :MAX_OUT, :C], orow[:, :MAX_OUT, 4:5]], axis=-1)
    return boxes_out, scores_out, ids_out, logits_out
